# baseline (speedup 1.0000x reference)
import math

import jax
import jax.numpy as jnp
import numpy as np
from jax.experimental import pallas as pl
from jax.experimental.pallas import tpu as pltpu

MAX_L = 8
K = (MAX_L + 1) ** 2
LANES = 128
CHUNK = 64
SUBS = 4
NBUF = 2
SQRT2 = math.sqrt(2.0)
Y00 = math.sqrt(1.0 / (4.0 * math.pi))
NBITS = 4


def _build_lane_consts():
    l_of = np.zeros(LANES, np.int64)
    m_of = np.zeros(LANES, np.int64)
    for l in range(MAX_L + 1):
        for m in range(-l, l + 1):
            l_of[l * (l + 1) + m] = l
            m_of[l * (l + 1) + m] = m
    am = np.abs(m_of)

    pref = np.zeros(LANES, np.float64)
    for j in range(LANES):
        p = Y00
        for q in range(1, am[j] + 1):
            p *= -math.sqrt((2 * q + 1) / (2.0 * q))
        pref[j] = p

    A = np.zeros((MAX_L + 1, LANES), np.float64)
    B = np.zeros((MAX_L + 1, LANES), np.float64)
    for l in range(1, MAX_L + 1):
        for j in range(LANES):
            a = am[j]
            if a == l - 1:
                A[l, j] = math.sqrt(2 * l + 1)
            elif a <= l - 2:
                A[l, j] = math.sqrt((4.0 * l * l - 1.0) / (l * l - a * a))
                B[l, j] = -math.sqrt(((2 * l + 1.0) * ((l - 1) ** 2 - a * a))
                                     / ((2 * l - 3.0) * (l * l - a * a)))

    rows = []
    rows.append(pref)
    rows.append(np.where(m_of != 0, SQRT2, 1.0))
    rows.append((m_of > 0).astype(np.float64))
    rows.append((m_of < 0).astype(np.float64))
    for k in range(NBITS):
        rows.append(((am >> k) & 1).astype(np.float64))
    for l in range(MAX_L + 1):
        rows.append((am == l).astype(np.float64))
    for l in range(1, MAX_L + 1):
        rows.append((l_of == l).astype(np.float64))
    for l in range(1, MAX_L + 1):
        rows.append(A[l])
    for l in range(2, MAX_L + 1):
        rows.append(B[l])
    return np.asarray(rows, np.float32)


_LANE_TBL = _build_lane_consts()


def _unpack_tbl(tb):
    row = lambda r: tb[r:r + 1, :]
    msk = lambda r: tb[r:r + 1, :] > 0.5
    c = {}
    c['pref'] = row(0)
    c['tv'] = row(1)
    c['mpos'] = msk(2)
    c['mneg'] = msk(3)
    c['bit'] = [msk(4 + k) for k in range(NBITS)]
    c['seedm'] = [msk(8 + l) for l in range(MAX_L + 1)]
    c['selm'] = [None] + [msk(17 + l - 1) for l in range(1, MAX_L + 1)]
    c['A'] = [None] + [row(25 + l - 1) for l in range(1, MAX_L + 1)]
    c['B'] = [None, None] + [row(33 + l - 2) for l in range(2, MAX_L + 1)]
    return c


def _chain_lane(x, y, z, C):
    rho2 = x * x + y * y
    r2 = rho2 + z * z
    ct = jnp.clip(z * jax.lax.rsqrt(r2), -1.0, 1.0)
    st = jnp.sqrt(jnp.maximum(1.0 - ct * ct, 0.0))
    safe = rho2 > 0.0
    inv_rho = jax.lax.rsqrt(jnp.where(safe, rho2, 1.0))
    ca = jnp.where(safe, x * inv_rho, 1.0)
    sa = jnp.where(safe, y * inv_rho, 0.0)

    pow_c = {0: (ca, sa)}
    for k in range(1, NBITS):
        cp, sp = pow_c[k - 1]
        t = cp * sp
        pow_c[k] = (cp * cp - sp * sp, t + t)
    cc = jnp.where(C['bit'][0], ca, 1.0)
    ss = jnp.where(C['bit'][0], sa, 0.0)
    for k in range(1, NBITS):
        ck, sk = pow_c[k]
        cn = cc * ck - ss * sk
        sn = ss * ck + cc * sk
        cc = jnp.where(C['bit'][k], cn, cc)
        ss = jnp.where(C['bit'][k], sn, ss)
    trig = jnp.where(C['mpos'], cc, jnp.where(C['mneg'], ss, 1.0)) * C['tv']

    stp = {0: st}
    for k in range(1, NBITS):
        stp[k] = stp[k - 1] * stp[k - 1]
    r = jnp.where(C['bit'][0], st, 1.0)
    for k in range(1, NBITS):
        r = jnp.where(C['bit'][k], r * stp[k], r)
    seed = C['pref'] * r

    psi = None
    p_prev = p_prev2 = None
    for l in range(MAX_L + 1):
        if l == 0:
            p_l = jnp.where(C['seedm'][0], seed, 0.0)
        else:
            w = C['A'][l] * (ct * p_prev)
            if l >= 2:
                w = w + C['B'][l] * p_prev2
            p_l = jnp.where(C['seedm'][l], seed, w)
        psi = p_l if l == 0 else jnp.where(C['selm'][l], p_l, psi)
        p_prev2, p_prev = p_prev, p_l
    return psi * trig


def _compute_chunk(c_rows, out_view, C):
    rep = [jnp.broadcast_to(c_rows[:, k:k + 1], (CHUNK, LANES))
           for k in range(6)]
    psi1 = _chain_lane(rep[0], rep[1], rep[2], C)
    psi2 = _chain_lane(rep[3], rep[4], rep[5], C)[:, :K]
    for i in range(K):
        col = jnp.broadcast_to(psi1[:, i:i + 1], (CHUNK, K))
        out_view[:, i * K:(i + 1) * K] = col * psi2


def _tph_kernel(c_ref, t_ref, o_ref, scr, sem):
    pid = pl.program_id(0)
    rows_per_core = c_ref.shape[0]
    n_sub = rows_per_core // CHUNK
    dma_rows = SUBS * CHUNK
    core_base = pid * rows_per_core

    def body(k, carry):
        sub = jax.lax.rem(k, SUBS)
        sup = jax.lax.div(k, SUBS)
        slot = jax.lax.rem(sup, NBUF)

        @pl.when((sub == 0) & (sup >= NBUF))
        def _():
            pltpu.make_async_copy(scr.at[slot], scr.at[slot],
                                  sem.at[slot]).wait()

        row0 = pl.multiple_of(sub * CHUNK, CHUNK)
        _compute_chunk(c_ref[pl.ds(k * CHUNK, CHUNK), :],
                       scr.at[slot].at[pl.ds(row0, CHUNK), :],
                       _unpack_tbl(t_ref[...]))

        @pl.when(sub == SUBS - 1)
        def _():
            dst = pl.ds(pl.multiple_of(core_base + sup * dma_rows, dma_rows),
                        dma_rows)
            pltpu.make_async_copy(scr.at[slot], o_ref.at[dst, :],
                                  sem.at[slot]).start()
        return carry

    jax.lax.fori_loop(0, n_sub, body, 0)
    for s in range(NBUF):
        pltpu.make_async_copy(scr.at[s], scr.at[s], sem.at[s]).wait()


def _tph_call(coordinates, interpret=False):
    n = coordinates.shape[0]
    return pl.pallas_call(
        _tph_kernel,
        grid=(2,),
        in_specs=[pl.BlockSpec((n // 2, 6), lambda c: (c, 0)),
                  pl.BlockSpec((40, LANES), lambda c: (0, 0))],
        out_specs=pl.BlockSpec(memory_space=pl.ANY),
        out_shape=jax.ShapeDtypeStruct((n, K * K), jnp.float32),
        scratch_shapes=[
            pltpu.VMEM((NBUF, SUBS * CHUNK, K * K), jnp.float32),
            pltpu.SemaphoreType.DMA((NBUF,)),
        ],
        compiler_params=pltpu.CompilerParams(
            dimension_semantics=("parallel",),
            vmem_limit_bytes=56 * 1024 * 1024,
        ),
        interpret=interpret,
    )(coordinates, jnp.asarray(_LANE_TBL))


@jax.jit
def kernel(coordinates):
    return _tph_call(coordinates)


# speedup vs baseline: 1.9427x; 1.9427x over previous
import math

import jax
import jax.numpy as jnp
import numpy as np
from jax.experimental import pallas as pl
from jax.experimental.pallas import tpu as pltpu

MAX_L = 8
K = (MAX_L + 1) ** 2
LANES = 128
CHUNK = 256
SUBS = 1
NBUF = 2
SQRT2 = math.sqrt(2.0)
Y00 = math.sqrt(1.0 / (4.0 * math.pi))
NBITS = 4


def _build_lane_consts():
    l_of = np.zeros(LANES, np.int64)
    m_of = np.zeros(LANES, np.int64)
    for l in range(MAX_L + 1):
        for m in range(-l, l + 1):
            l_of[l * (l + 1) + m] = l
            m_of[l * (l + 1) + m] = m
    am = np.abs(m_of)

    pref = np.zeros(LANES, np.float64)
    for j in range(LANES):
        p = Y00
        for q in range(1, am[j] + 1):
            p *= -math.sqrt((2 * q + 1) / (2.0 * q))
        pref[j] = p

    A = np.zeros((MAX_L + 1, LANES), np.float64)
    B = np.zeros((MAX_L + 1, LANES), np.float64)
    for l in range(1, MAX_L + 1):
        for j in range(LANES):
            a = am[j]
            if a == l - 1:
                A[l, j] = math.sqrt(2 * l + 1)
            elif a <= l - 2:
                A[l, j] = math.sqrt((4.0 * l * l - 1.0) / (l * l - a * a))
                B[l, j] = -math.sqrt(((2 * l + 1.0) * ((l - 1) ** 2 - a * a))
                                     / ((2 * l - 3.0) * (l * l - a * a)))

    rows = []
    rows.append(pref)
    rows.append(np.where(m_of != 0, SQRT2, 1.0))
    rows.append((m_of > 0).astype(np.float64))
    rows.append((m_of < 0).astype(np.float64))
    for k in range(NBITS):
        rows.append(((am >> k) & 1).astype(np.float64))
    for l in range(MAX_L + 1):
        rows.append((am == l).astype(np.float64))
    for l in range(1, MAX_L + 1):
        rows.append((l_of == l).astype(np.float64))
    for l in range(1, MAX_L + 1):
        rows.append(A[l])
    for l in range(2, MAX_L + 1):
        rows.append(B[l])
    return np.asarray(rows, np.float32)


_LANE_TBL = _build_lane_consts()


def _unpack_tbl(tb):
    row = lambda r: tb[r:r + 1, :]
    msk = lambda r: tb[r:r + 1, :] > 0.5
    c = {}
    c['pref'] = row(0)
    c['tv'] = row(1)
    c['mpos'] = msk(2)
    c['mneg'] = msk(3)
    c['bit'] = [msk(4 + k) for k in range(NBITS)]
    c['seedm'] = [msk(8 + l) for l in range(MAX_L + 1)]
    c['selm'] = [None] + [msk(17 + l - 1) for l in range(1, MAX_L + 1)]
    c['A'] = [None] + [row(25 + l - 1) for l in range(1, MAX_L + 1)]
    c['B'] = [None, None] + [row(33 + l - 2) for l in range(2, MAX_L + 1)]
    return c


def _chain_lane(x, y, z, C):
    rho2 = x * x + y * y
    r2 = rho2 + z * z
    ct = jnp.clip(z * jax.lax.rsqrt(r2), -1.0, 1.0)
    st = jnp.sqrt(jnp.maximum(1.0 - ct * ct, 0.0))
    safe = rho2 > 0.0
    inv_rho = jax.lax.rsqrt(jnp.where(safe, rho2, 1.0))
    ca = jnp.where(safe, x * inv_rho, 1.0)
    sa = jnp.where(safe, y * inv_rho, 0.0)

    pow_c = {0: (ca, sa)}
    for k in range(1, NBITS):
        cp, sp = pow_c[k - 1]
        t = cp * sp
        pow_c[k] = (cp * cp - sp * sp, t + t)
    cc = jnp.where(C['bit'][0], ca, 1.0)
    ss = jnp.where(C['bit'][0], sa, 0.0)
    for k in range(1, NBITS):
        ck, sk = pow_c[k]
        cn = cc * ck - ss * sk
        sn = ss * ck + cc * sk
        cc = jnp.where(C['bit'][k], cn, cc)
        ss = jnp.where(C['bit'][k], sn, ss)
    trig = jnp.where(C['mpos'], cc, jnp.where(C['mneg'], ss, 1.0)) * C['tv']

    stp = {0: st}
    for k in range(1, NBITS):
        stp[k] = stp[k - 1] * stp[k - 1]
    r = jnp.where(C['bit'][0], st, 1.0)
    for k in range(1, NBITS):
        r = jnp.where(C['bit'][k], r * stp[k], r)
    seed = C['pref'] * r

    psi = None
    p_prev = p_prev2 = None
    for l in range(MAX_L + 1):
        if l == 0:
            p_l = jnp.where(C['seedm'][0], seed, 0.0)
        else:
            w = C['A'][l] * (ct * p_prev)
            if l >= 2:
                w = w + C['B'][l] * p_prev2
            p_l = jnp.where(C['seedm'][l], seed, w)
        psi = p_l if l == 0 else jnp.where(C['selm'][l], p_l, psi)
        p_prev2, p_prev = p_prev, p_l
    return psi * trig


def _compute_chunk(c_rows, out_view, C):
    rep = [jnp.broadcast_to(c_rows[:, k:k + 1], (CHUNK, LANES))
           for k in range(6)]
    psi1 = _chain_lane(rep[0], rep[1], rep[2], C)
    psi2 = _chain_lane(rep[3], rep[4], rep[5], C)[:, :K]
    for i in range(K):
        col = jnp.broadcast_to(psi1[:, i:i + 1], (CHUNK, K))
        out_view[:, i * K:(i + 1) * K] = col * psi2


def _tph_kernel(c_ref, t_ref, o_ref, scr, sem):
    pid = pl.program_id(0)
    rows_per_core = c_ref.shape[0]
    n_sub = rows_per_core // CHUNK
    dma_rows = SUBS * CHUNK
    core_base = pid * rows_per_core

    def body(k, carry):
        sub = jax.lax.rem(k, SUBS)
        sup = jax.lax.div(k, SUBS)
        slot = jax.lax.rem(sup, NBUF)

        @pl.when((sub == 0) & (sup >= NBUF))
        def _():
            pltpu.make_async_copy(scr.at[slot], scr.at[slot],
                                  sem.at[slot]).wait()

        row0 = pl.multiple_of(sub * CHUNK, CHUNK)
        _compute_chunk(c_ref[pl.ds(k * CHUNK, CHUNK), :],
                       scr.at[slot].at[pl.ds(row0, CHUNK), :],
                       _unpack_tbl(t_ref[...]))

        @pl.when(sub == SUBS - 1)
        def _():
            dst = pl.ds(pl.multiple_of(core_base + sup * dma_rows, dma_rows),
                        dma_rows)
            pltpu.make_async_copy(scr.at[slot], o_ref.at[dst, :],
                                  sem.at[slot]).start()
        return carry

    jax.lax.fori_loop(0, n_sub, body, 0)
    for s in range(NBUF):
        pltpu.make_async_copy(scr.at[s], scr.at[s], sem.at[s]).wait()


def _tph_call(coordinates, interpret=False):
    n = coordinates.shape[0]
    return pl.pallas_call(
        _tph_kernel,
        grid=(2,),
        in_specs=[pl.BlockSpec((n // 2, 6), lambda c: (c, 0)),
                  pl.BlockSpec((40, LANES), lambda c: (0, 0))],
        out_specs=pl.BlockSpec(memory_space=pl.ANY),
        out_shape=jax.ShapeDtypeStruct((n, K * K), jnp.float32),
        scratch_shapes=[
            pltpu.VMEM((NBUF, SUBS * CHUNK, K * K), jnp.float32),
            pltpu.SemaphoreType.DMA((NBUF,)),
        ],
        compiler_params=pltpu.CompilerParams(
            dimension_semantics=("parallel",),
            vmem_limit_bytes=56 * 1024 * 1024,
        ),
        interpret=interpret,
    )(coordinates, jnp.asarray(_LANE_TBL))


@jax.jit
def kernel(coordinates):
    return _tph_call(coordinates)


# speedup vs baseline: 3.1316x; 1.6120x over previous
import math

import jax
import jax.numpy as jnp
import numpy as np
from jax.experimental import pallas as pl
from jax.experimental.pallas import tpu as pltpu

MAX_L = 8
K = (MAX_L + 1) ** 2
LANES = 128
CHUNK = 256
SUBS = 1
NBUF = 2
SQRT2 = math.sqrt(2.0)
Y00 = math.sqrt(1.0 / (4.0 * math.pi))
NBITS = 4


def _build_lane_consts():
    l_of = np.zeros(LANES, np.int64)
    m_of = np.zeros(LANES, np.int64)
    for l in range(MAX_L + 1):
        for m in range(-l, l + 1):
            l_of[l * (l + 1) + m] = l
            m_of[l * (l + 1) + m] = m
    am = np.abs(m_of)

    pref = np.zeros(LANES, np.float64)
    for j in range(LANES):
        p = Y00
        for q in range(1, am[j] + 1):
            p *= -math.sqrt((2 * q + 1) / (2.0 * q))
        pref[j] = p

    A = np.zeros((MAX_L + 1, LANES), np.float64)
    B = np.zeros((MAX_L + 1, LANES), np.float64)
    for l in range(1, MAX_L + 1):
        for j in range(LANES):
            a = am[j]
            if a == l - 1:
                A[l, j] = math.sqrt(2 * l + 1)
            elif a <= l - 2:
                A[l, j] = math.sqrt((4.0 * l * l - 1.0) / (l * l - a * a))
                B[l, j] = -math.sqrt(((2 * l + 1.0) * ((l - 1) ** 2 - a * a))
                                     / ((2 * l - 3.0) * (l * l - a * a)))

    rows = []
    rows.append(pref)
    rows.append(np.where(m_of != 0, SQRT2, 1.0))
    rows.append((m_of > 0).astype(np.float64))
    rows.append((m_of < 0).astype(np.float64))
    for k in range(NBITS):
        rows.append(((am >> k) & 1).astype(np.float64))
    for l in range(MAX_L + 1):
        rows.append((am == l).astype(np.float64))
    for l in range(1, MAX_L + 1):
        rows.append((l_of == l).astype(np.float64))
    for l in range(1, MAX_L + 1):
        rows.append(A[l])
    for l in range(2, MAX_L + 1):
        rows.append(B[l])
    return np.asarray(rows, np.float32)


_LANE_TBL = _build_lane_consts()


def _unpack_tbl(tb):
    row = lambda r: tb[r:r + 1, :]
    msk = lambda r: tb[r:r + 1, :] > 0.5
    c = {}
    c['pref'] = row(0)
    c['tv'] = row(1)
    c['mpos'] = msk(2)
    c['mneg'] = msk(3)
    c['bit'] = [msk(4 + k) for k in range(NBITS)]
    c['seedm'] = [msk(8 + l) for l in range(MAX_L + 1)]
    c['selm'] = [None] + [msk(17 + l - 1) for l in range(1, MAX_L + 1)]
    c['A'] = [None] + [row(25 + l - 1) for l in range(1, MAX_L + 1)]
    c['B'] = [None, None] + [row(33 + l - 2) for l in range(2, MAX_L + 1)]
    return c


def _chain_lane(x, y, z, C):
    rho2 = x * x + y * y
    r2 = rho2 + z * z
    ct = jnp.clip(z * jax.lax.rsqrt(r2), -1.0, 1.0)
    st = jnp.sqrt(jnp.maximum(1.0 - ct * ct, 0.0))
    safe = rho2 > 0.0
    inv_rho = jax.lax.rsqrt(jnp.where(safe, rho2, 1.0))
    ca = jnp.where(safe, x * inv_rho, 1.0)
    sa = jnp.where(safe, y * inv_rho, 0.0)

    pow_c = {0: (ca, sa)}
    for k in range(1, NBITS):
        cp, sp = pow_c[k - 1]
        t = cp * sp
        pow_c[k] = (cp * cp - sp * sp, t + t)
    cc = jnp.where(C['bit'][0], ca, 1.0)
    ss = jnp.where(C['bit'][0], sa, 0.0)
    for k in range(1, NBITS):
        ck, sk = pow_c[k]
        cn = cc * ck - ss * sk
        sn = ss * ck + cc * sk
        cc = jnp.where(C['bit'][k], cn, cc)
        ss = jnp.where(C['bit'][k], sn, ss)
    trig = jnp.where(C['mpos'], cc, jnp.where(C['mneg'], ss, 1.0)) * C['tv']

    stp = {0: st}
    for k in range(1, NBITS):
        stp[k] = stp[k - 1] * stp[k - 1]
    r = jnp.where(C['bit'][0], st, 1.0)
    for k in range(1, NBITS):
        r = jnp.where(C['bit'][k], r * stp[k], r)
    seed = C['pref'] * r

    psi = None
    p_prev = p_prev2 = None
    for l in range(MAX_L + 1):
        if l == 0:
            p_l = jnp.where(C['seedm'][0], seed, 0.0)
        else:
            w = C['A'][l] * (ct * p_prev)
            if l >= 2:
                w = w + C['B'][l] * p_prev2
            p_l = jnp.where(C['seedm'][l], seed, w)
        psi = p_l if l == 0 else jnp.where(C['selm'][l], p_l, psi)
        p_prev2, p_prev = p_prev, p_l
    return psi * trig


def _chain(x, y, z, emit):
    rho2 = x * x + y * y
    r2 = rho2 + z * z
    ct = jnp.clip(z * jax.lax.rsqrt(r2), -1.0, 1.0)
    st = jnp.sqrt(jnp.maximum(1.0 - ct * ct, 0.0))
    safe = rho2 > 0.0
    inv_rho = jax.lax.rsqrt(jnp.where(safe, rho2, 1.0))
    ca = jnp.where(safe, x * inv_rho, 1.0)
    sa = jnp.where(safe, y * inv_rho, 0.0)

    pmm = jnp.full_like(x, Y00)
    cmv = smv = None
    c2m = s2m = None
    for m in range(MAX_L + 1):
        if m > 0:
            pmm = (-math.sqrt((2 * m + 1) / (2.0 * m)) * st) * pmm
            if m == 1:
                cmv, smv = ca, sa
            else:
                cmv, smv = cmv * ca - smv * sa, smv * ca + cmv * sa
            c2m = SQRT2 * cmv
            s2m = SQRT2 * smv

        def em(l, p):
            if m == 0:
                emit(l * (l + 1), p)
            else:
                emit(l * (l + 1) + m, p * c2m)
                emit(l * (l + 1) - m, p * s2m)

        em(m, pmm)
        if m < MAX_L:
            p_prev2 = pmm
            p_prev = (math.sqrt(2 * m + 3) * ct) * pmm
            em(m + 1, p_prev)
            for l in range(m + 2, MAX_L + 1):
                a = math.sqrt((4.0 * l * l - 1.0) / (l * l - m * m))
                b = -math.sqrt(((2 * l + 1.0) * ((l - 1) ** 2 - m * m))
                               / ((2 * l - 3.0) * (l * l - m * m)))
                p = a * (ct * p_prev) + b * p_prev2
                em(l, p)
                p_prev2, p_prev = p_prev, p


def _compute_chunk(c_rows, out_view, C):
    rep = [jnp.broadcast_to(c_rows[:, k:k + 1], (CHUNK, LANES))
           for k in range(6)]
    psi2 = _chain_lane(rep[3], rep[4], rep[5], C)[:, :K]

    def emit1(idx, col):
        out_view[:, idx * K:(idx + 1) * K] = col[:, :K] * psi2

    _chain(rep[0], rep[1], rep[2], emit1)


def _tph_kernel(c_ref, t_ref, o_ref, scr, sem):
    pid = pl.program_id(0)
    rows_per_core = c_ref.shape[0]
    n_sub = rows_per_core // CHUNK
    dma_rows = SUBS * CHUNK
    core_base = pid * rows_per_core

    def body(k, carry):
        sub = jax.lax.rem(k, SUBS)
        sup = jax.lax.div(k, SUBS)
        slot = jax.lax.rem(sup, NBUF)

        @pl.when((sub == 0) & (sup >= NBUF))
        def _():
            pltpu.make_async_copy(scr.at[slot], scr.at[slot],
                                  sem.at[slot]).wait()

        row0 = pl.multiple_of(sub * CHUNK, CHUNK)
        _compute_chunk(c_ref[pl.ds(k * CHUNK, CHUNK), :],
                       scr.at[slot].at[pl.ds(row0, CHUNK), :],
                       _unpack_tbl(t_ref[...]))

        @pl.when(sub == SUBS - 1)
        def _():
            dst = pl.ds(pl.multiple_of(core_base + sup * dma_rows, dma_rows),
                        dma_rows)
            pltpu.make_async_copy(scr.at[slot], o_ref.at[dst, :],
                                  sem.at[slot]).start()
        return carry

    jax.lax.fori_loop(0, n_sub, body, 0)
    for s in range(NBUF):
        pltpu.make_async_copy(scr.at[s], scr.at[s], sem.at[s]).wait()


def _tph_call(coordinates, interpret=False):
    n = coordinates.shape[0]
    return pl.pallas_call(
        _tph_kernel,
        grid=(2,),
        in_specs=[pl.BlockSpec((n // 2, 6), lambda c: (c, 0)),
                  pl.BlockSpec((40, LANES), lambda c: (0, 0))],
        out_specs=pl.BlockSpec(memory_space=pl.ANY),
        out_shape=jax.ShapeDtypeStruct((n, K * K), jnp.float32),
        scratch_shapes=[
            pltpu.VMEM((NBUF, SUBS * CHUNK, K * K), jnp.float32),
            pltpu.SemaphoreType.DMA((NBUF,)),
        ],
        compiler_params=pltpu.CompilerParams(
            dimension_semantics=("parallel",),
            vmem_limit_bytes=56 * 1024 * 1024,
        ),
        interpret=interpret,
    )(coordinates, jnp.asarray(_LANE_TBL))


@jax.jit
def kernel(coordinates):
    return _tph_call(coordinates)
